# baseline (speedup 1.0000x reference)
import functools

import jax
import jax.numpy as jnp
from jax import lax
from jax.experimental import pallas as pl
from jax.experimental.pallas import tpu as pltpu
from jax.experimental.pallas import tpu_sc as plsc

_B = 4096
_DU = 128
_DA = 32
_L = 16


@functools.lru_cache(maxsize=None)
def _build():
    info = plsc.get_sparse_core_info()
    nc, ns = info.num_cores, info.num_subcores
    nw = nc * ns
    bpw = _B // nw

    mesh = plsc.VectorSubcoreMesh(core_axis_name="c", subcore_axis_name="s")

    @functools.partial(
        pl.kernel,
        mesh=mesh,
        compiler_params=pltpu.CompilerParams(needs_layout_passes=False),
        out_type=(
            jax.ShapeDtypeStruct((_B, _DU), jnp.float32),
            jax.ShapeDtypeStruct((_B, _DU), jnp.float32),
            jax.ShapeDtypeStruct((_B, _DA), jnp.float32),
        ),
        scratch_types=[
            pltpu.VMEM((bpw,), jnp.int32),
            pltpu.VMEM((bpw,), jnp.int32),
            pltpu.VMEM((bpw,), jnp.int32),
            pltpu.VMEM((bpw,), jnp.int32),
            pltpu.VMEM((bpw, _DU), jnp.float32),
            pltpu.VMEM((bpw, _DU), jnp.float32),
            pltpu.VMEM((bpw, _DU), jnp.float32),
            pltpu.VMEM((bpw, _DA), jnp.float32),
            pltpu.SemaphoreType.DMA,
            pltpu.SemaphoreType.DMA,
            pltpu.SemaphoreType.DMA,
        ],
    )
    def emb(user_hbm, item_hbm, attrw_hbm, un_hbm, in_hbm, an_hbm,
            u_out, i_out, a_out,
            uidx, iidx, aidx, aidx4, urows, irows, awide, arows, su, si, sa):
        wid = lax.axis_index("s") * nc + lax.axis_index("c")
        base = wid * bpw
        pltpu.sync_copy(un_hbm.at[pl.ds(base, bpw)], uidx)
        pltpu.sync_copy(in_hbm.at[pl.ds(base, bpw)], iidx)
        pltpu.sync_copy(an_hbm.at[pl.ds(base, bpw)], aidx)
        cu = pltpu.async_copy(user_hbm.at[uidx], urows, su)
        ci = pltpu.async_copy(item_hbm.at[iidx], irows, si)
        for j in range(bpw // _L):
            aidx4[pl.ds(j * _L, _L)] = lax.shift_right_logical(
                aidx[pl.ds(j * _L, _L)], 2)
        ca = pltpu.async_copy(attrw_hbm.at[aidx4], awide, sa)
        ca.wait()
        iota = lax.iota(jnp.int32, _L)

        def extract(b, _):
            bvec = jnp.full((_L,), b, jnp.int32)
            iv = plsc.load_gather(aidx, [bvec])
            col = (iv & 3) * _DA + iota
            arows[b, pl.ds(0, _L)] = plsc.load_gather(awide, [bvec, col])
            arows[b, pl.ds(_L, _L)] = plsc.load_gather(
                awide, [bvec, col + _L])
            return _

        lax.fori_loop(0, bpw, extract, None)
        pltpu.sync_copy(arows, a_out.at[pl.ds(base, bpw)])
        cu.wait()
        pltpu.sync_copy(urows, u_out.at[pl.ds(base, bpw)])
        ci.wait()
        pltpu.sync_copy(irows, i_out.at[pl.ds(base, bpw)])

    return emb


def kernel(user_table, item_table, attr_table, user_nodes, item_nodes,
           attribute_nodes):
    emb = _build()
    attr_wide = attr_table.reshape(attr_table.shape[0] * _DA // _DU, _DU)
    return emb(
        user_table, item_table, attr_wide,
        user_nodes.astype(jnp.int32),
        item_nodes.astype(jnp.int32),
        attribute_nodes.astype(jnp.int32),
    )


# speedup vs baseline: 1.0504x; 1.0504x over previous
import functools

import jax
import jax.numpy as jnp
from jax import lax
from jax.experimental import pallas as pl
from jax.experimental.pallas import tpu as pltpu
from jax.experimental.pallas import tpu_sc as plsc

_B = 4096
_DU = 128
_DA = 32
_L = 16


@functools.lru_cache(maxsize=None)
def _build():
    info = plsc.get_sparse_core_info()
    nc, ns = info.num_cores, info.num_subcores
    nw = nc * ns
    bpw = _B // nw

    mesh = plsc.VectorSubcoreMesh(core_axis_name="c", subcore_axis_name="s")

    @functools.partial(
        pl.kernel,
        mesh=mesh,
        compiler_params=pltpu.CompilerParams(needs_layout_passes=False),
        out_type=(
            jax.ShapeDtypeStruct((_B, _DU), jnp.float32),
            jax.ShapeDtypeStruct((_B, _DU), jnp.float32),
            jax.ShapeDtypeStruct((_DA, _B), jnp.float32),
        ),
        scratch_types=[
            pltpu.VMEM((bpw,), jnp.int32),
            pltpu.VMEM((bpw,), jnp.int32),
            pltpu.VMEM((bpw,), jnp.int32),
            pltpu.VMEM((bpw,), jnp.int32),
            pltpu.VMEM((bpw, _DU), jnp.float32),
            pltpu.VMEM((bpw, _DU), jnp.float32),
            pltpu.VMEM((bpw, _DU), jnp.float32),
            pltpu.VMEM((_DA, bpw), jnp.float32),
            pltpu.SemaphoreType.DMA,
            pltpu.SemaphoreType.DMA,
            pltpu.SemaphoreType.DMA,
            pltpu.SemaphoreType.DMA,
            pltpu.SemaphoreType.DMA,
            pltpu.SemaphoreType.DMA,
        ],
    )
    def emb(user_hbm, item_hbm, attrw_hbm, un_hbm, in_hbm, an_hbm,
            u_out, i_out, a_out,
            uidx, iidx, aidx, aidx4, urows, irows, awide, arowst,
            su, si, sa, syu, syi, sya):
        wid = lax.axis_index("s") * nc + lax.axis_index("c")
        base = wid * bpw
        ga = pltpu.async_copy(an_hbm.at[pl.ds(base, bpw)], aidx, sya)
        gu = pltpu.async_copy(un_hbm.at[pl.ds(base, bpw)], uidx, syu)
        gi = pltpu.async_copy(in_hbm.at[pl.ds(base, bpw)], iidx, syi)
        ga.wait()
        for j in range(bpw // _L):
            aidx4[pl.ds(j * _L, _L)] = lax.shift_right_logical(
                aidx[pl.ds(j * _L, _L)], 2)
        ca = pltpu.async_copy(attrw_hbm.at[aidx4], awide, sa)
        gu.wait()
        cu = pltpu.async_copy(user_hbm.at[uidx], urows, su)
        gi.wait()
        ci = pltpu.async_copy(item_hbm.at[iidx], irows, si)
        ca.wait()
        iota = lax.iota(jnp.int32, _L)
        for j in range(bpw // _L):
            rows = iota + (j * _L)
            col0 = (aidx[pl.ds(j * _L, _L)] & 3) * _DA
            for c in range(_DA):
                arowst[c, pl.ds(j * _L, _L)] = plsc.load_gather(
                    awide, [rows, col0 + c])
        pltpu.sync_copy(arowst, a_out.at[:, pl.ds(base, bpw)])
        cu.wait()
        pltpu.sync_copy(urows, u_out.at[pl.ds(base, bpw)])
        ci.wait()
        pltpu.sync_copy(irows, i_out.at[pl.ds(base, bpw)])

    return emb


def kernel(user_table, item_table, attr_table, user_nodes, item_nodes,
           attribute_nodes):
    emb = _build()
    attr_wide = attr_table.reshape(attr_table.shape[0] * _DA // _DU, _DU)
    u, i, at = emb(
        user_table, item_table, attr_wide,
        user_nodes.astype(jnp.int32),
        item_nodes.astype(jnp.int32),
        attribute_nodes.astype(jnp.int32),
    )
    return (u, i, at.T)


# speedup vs baseline: 1.0630x; 1.0120x over previous
import functools

import jax
import jax.numpy as jnp
from jax import lax
from jax.experimental import pallas as pl
from jax.experimental.pallas import tpu as pltpu
from jax.experimental.pallas import tpu_sc as plsc

_B = 4096
_DU = 128
_DA = 32
_L = 16


@functools.lru_cache(maxsize=None)
def _build():
    info = plsc.get_sparse_core_info()
    nc, ns = info.num_cores, info.num_subcores
    nw = nc * ns
    bpw = _B // nw

    mesh = plsc.VectorSubcoreMesh(core_axis_name="c", subcore_axis_name="s")

    @functools.partial(
        pl.kernel,
        mesh=mesh,
        compiler_params=pltpu.CompilerParams(needs_layout_passes=False),
        out_type=(
            jax.ShapeDtypeStruct((_B, _DU), jnp.float32),
            jax.ShapeDtypeStruct((_B, _DU), jnp.float32),
            jax.ShapeDtypeStruct((_DA, _B), jnp.float32),
        ),
        scratch_types=[
            pltpu.VMEM((bpw,), jnp.int32),
            pltpu.VMEM((bpw,), jnp.int32),
            pltpu.VMEM((bpw,), jnp.int32),
            pltpu.VMEM((bpw,), jnp.int32),
            pltpu.VMEM((bpw, _DU), jnp.float32),
            pltpu.VMEM((bpw, _DU), jnp.float32),
            pltpu.VMEM((bpw, _DU), jnp.float32),
            pltpu.VMEM((_DA, bpw), jnp.float32),
            pltpu.SemaphoreType.DMA,
            pltpu.SemaphoreType.DMA,
            pltpu.SemaphoreType.DMA,
            pltpu.SemaphoreType.DMA,
            pltpu.SemaphoreType.DMA,
            pltpu.SemaphoreType.DMA,
        ],
    )
    def emb(user_hbm, item_hbm, attr_hbm, un_hbm, in_hbm, an_hbm,
            u_out, i_out, a_out,
            uidx, iidx, aidx, aidx4, urows, irows, awide, arowst,
            su, si, sa, syu, syi, sya):
        attrw_hbm = attr_hbm.reshape(1000 * _DA // _DU, _DU)
        wid = lax.axis_index("s") * nc + lax.axis_index("c")
        base = wid * bpw
        ga = pltpu.async_copy(an_hbm.at[pl.ds(base, bpw)], aidx, sya)
        gu = pltpu.async_copy(un_hbm.at[pl.ds(base, bpw)], uidx, syu)
        gi = pltpu.async_copy(in_hbm.at[pl.ds(base, bpw)], iidx, syi)
        ga.wait()
        for j in range(bpw // _L):
            aidx4[pl.ds(j * _L, _L)] = lax.shift_right_logical(
                aidx[pl.ds(j * _L, _L)], 2)
        ca = pltpu.async_copy(attrw_hbm.at[aidx4], awide, sa)
        gu.wait()
        cu = pltpu.async_copy(user_hbm.at[uidx], urows, su)
        gi.wait()
        ci = pltpu.async_copy(item_hbm.at[iidx], irows, si)
        ca.wait()
        iota = lax.iota(jnp.int32, _L)
        for j in range(bpw // _L):
            rows = iota + (j * _L)
            col0 = (aidx[pl.ds(j * _L, _L)] & 3) * _DA
            for c in range(_DA):
                arowst[c, pl.ds(j * _L, _L)] = plsc.load_gather(
                    awide, [rows, col0 + c])
        pltpu.sync_copy(arowst, a_out.at[:, pl.ds(base, bpw)])
        cu.wait()
        pltpu.sync_copy(urows, u_out.at[pl.ds(base, bpw)])
        ci.wait()
        pltpu.sync_copy(irows, i_out.at[pl.ds(base, bpw)])

    return emb


def kernel(user_table, item_table, attr_table, user_nodes, item_nodes,
           attribute_nodes):
    emb = _build()
    attr_wide = attr_table.reshape(attr_table.shape[0] * _DA // _DU, _DU)
    u, i, at = emb(
        user_table, item_table, attr_wide,
        user_nodes.astype(jnp.int32),
        item_nodes.astype(jnp.int32),
        attribute_nodes.astype(jnp.int32),
    )
    return (u, i, at.T)
